# baseline (speedup 1.0000x reference)
import functools

import jax
import jax.numpy as jnp
from jax import lax
from jax.experimental import pallas as pl
from jax.experimental.pallas import tpu as pltpu
from jax.experimental.pallas import tpu_sc as plsc

_F32 = jnp.float32
_PI = 3.141592653589793

_C = 96
_HW = 384 * 384
_NPIX = 2 * _HW
_NW = 32
_PPW = _NPIX // _NW
_WC = 256
_CHUNKS = _PPW // _WC
_CUNROLL = 12


def _sc_stats(o2, l2):
    mesh = plsc.VectorSubcoreMesh(core_axis_name="c", subcore_axis_name="s")
    out_t = tuple(jax.ShapeDtypeStruct((_NPIX,), _F32) for _ in range(4))

    @functools.partial(
        pl.kernel,
        out_type=out_t,
        mesh=mesh,
        scratch_types=[
            pltpu.VMEM((2, _C, _WC), _F32),
            pltpu.VMEM((2, _C, _WC), _F32),
            pltpu.VMEM((2, 4, _WC), _F32),
            pltpu.SemaphoreType.DMA,
            pltpu.SemaphoreType.DMA,
        ],
    )
    def body(o_hbm, l_hbm, num_o, oo_o, ll_o, ls_o, obuf, lbuf, sbuf,
             isem, ssem):
        wid = lax.axis_index("s") * 2 + lax.axis_index("c")
        batch = wid // 16
        row0 = batch * _C
        pix0 = (wid % 16) * _PPW
        gbase = batch * _HW + pix0

        def in_copies(k, slot):
            p0 = pix0 + k * _WC
            src_o = o_hbm.at[pl.ds(row0, _C), pl.ds(p0, _WC)]
            src_l = l_hbm.at[pl.ds(row0, _C), pl.ds(p0, _WC)]
            return (pltpu.make_async_copy(src_o, obuf.at[slot], isem),
                    pltpu.make_async_copy(src_l, lbuf.at[slot], isem))

        def out_copies(k, slot):
            d0 = gbase + k * _WC
            return tuple(
                pltpu.make_async_copy(sbuf.at[slot, r],
                                      dst.at[pl.ds(d0, _WC)], ssem)
                for r, dst in enumerate((num_o, oo_o, ll_o, ls_o)))

        def compute(slot):
            def jbody(j, _):
                base = j * 16

                def cbody(ci, carry):
                    n, oo, llv, ls = carry
                    c0 = ci * _CUNROLL
                    for u in range(_CUNROLL):
                        ov = obuf[slot, c0 + u, pl.ds(base, 16)]
                        lv = lbuf[slot, c0 + u, pl.ds(base, 16)]
                        n = n + ov * lv
                        oo = oo + ov * ov
                        llv = llv + lv * lv
                        ls = ls + lv
                    return n, oo, llv, ls

                z = jnp.zeros((16,), _F32)
                n, oo, llv, ls = lax.fori_loop(
                    0, _C // _CUNROLL, cbody, (z, z, z, z))
                sbuf[slot, 0, pl.ds(base, 16)] = n
                sbuf[slot, 1, pl.ds(base, 16)] = oo
                sbuf[slot, 2, pl.ds(base, 16)] = llv
                sbuf[slot, 3, pl.ds(base, 16)] = ls
                return 0

            lax.fori_loop(0, _WC // 16, jbody, 0)

        for cp in in_copies(0, 0):
            cp.start()
        for k in range(_CHUNKS):
            slot = k & 1
            if k + 1 < _CHUNKS:
                for cp in in_copies(k + 1, 1 - slot):
                    cp.start()
            for cp in in_copies(k, slot):
                cp.wait()
            if k >= 2:
                for cp in out_copies(k - 2, slot):
                    cp.wait()
            compute(slot)
            for cp in out_copies(k, slot):
                cp.start()
        for k in (_CHUNKS - 2, _CHUNKS - 1):
            for cp in out_copies(k, k & 1):
                cp.wait()

    return body(o2, l2)


def _acos(x):
    ax = jnp.minimum(jnp.abs(x), 1.0)
    p = jnp.float32(-0.0012624911)
    for c in (0.0066700901, -0.0170881256, 0.0308918810, -0.0501743046,
              0.0889789874, -0.2145988016, 1.5707963050):
        p = p * ax + jnp.float32(c)
    r = jnp.sqrt(1.0 - ax) * p
    return jnp.where(x < 0, jnp.float32(_PI) - r, r)


def _finish_body(n_ref, oo_ref, ll_ref, ls_ref, out_ref, acc_ref):
    i = pl.program_id(0)
    num = n_ref[...]
    oo = oo_ref[...]
    ll = ll_ref[...]
    ls = ls_ref[...]
    mask = ls != 0.0
    den = jnp.sqrt(oo) * jnp.sqrt(ll)
    ratio = jnp.clip(num / jnp.where(mask, den, 1.0), -1.0, 1.0)
    ang = jnp.where(mask, _acos(ratio), 0.0)
    psum = jnp.sum(ang)
    pcnt = jnp.sum(mask.astype(_F32))

    @pl.when(i == 0)
    def _init():
        acc_ref[0] = 0.0
        acc_ref[1] = 0.0

    acc_ref[0] += psum
    acc_ref[1] += pcnt

    @pl.when(i == pl.num_programs(0) - 1)
    def _fin():
        out_ref[0, 0] = acc_ref[0] / acc_ref[1]


def kernel(outputs, labels):
    b, c, h, w = outputs.shape
    o2 = outputs.reshape(b * c, h * w)
    l2 = labels.reshape(b * c, h * w)
    num, oo, ll, ls = _sc_stats(o2, l2)
    rows = _NPIX // 128
    stats = [x.reshape(rows, 128) for x in (num, oo, ll, ls)]
    grid = 4
    spec = pl.BlockSpec((rows // grid, 128), lambda i: (i, 0))
    out = pl.pallas_call(
        _finish_body,
        grid=(grid,),
        in_specs=[spec] * 4,
        out_specs=pl.BlockSpec(memory_space=pltpu.SMEM),
        out_shape=jax.ShapeDtypeStruct((1, 1), _F32),
        scratch_shapes=[pltpu.SMEM((2,), _F32)],
    )(*stats)
    return out[0, 0]


# speedup vs baseline: 2.2221x; 2.2221x over previous
import functools

import jax
import jax.numpy as jnp
from jax import lax
from jax.experimental import pallas as pl
from jax.experimental.pallas import tpu as pltpu
from jax.experimental.pallas import tpu_sc as plsc

_F32 = jnp.float32
_PI = 3.141592653589793

_C = 96
_HW = 384 * 384
_NPIX = 2 * _HW
_NW = 32
_TILES = _NPIX // (8 * 128)
_JOBS = _TILES // _NW
_CQ = _C // 4


def _sc_stats(o3, l3):
    mesh = plsc.VectorSubcoreMesh(core_axis_name="c", subcore_axis_name="s")
    out_t = tuple(
        jax.ShapeDtypeStruct((_TILES * 8, 128), _F32) for _ in range(4))

    @functools.partial(
        pl.kernel,
        out_type=out_t,
        mesh=mesh,
        compiler_params=pltpu.CompilerParams(use_tc_tiling_on_sc=True),
        scratch_types=[
            pltpu.VMEM((2, _CQ, 8, 128), _F32),
            pltpu.VMEM((2, _CQ, 8, 128), _F32),
            pltpu.VMEM((4, 8, 128), _F32),
            pltpu.SemaphoreType.DMA,
            pltpu.SemaphoreType.DMA,
        ],
    )
    def body(o_hbm, l_hbm, num_o, oo_o, ll_o, ls_o, obuf, lbuf, sbuf,
             isem, ssem):
        wid = lax.axis_index("s") * 2 + lax.axis_index("c")
        outs = (num_o, oo_o, ll_o, ls_o)

        def in_copies(job, q, slot):
            t = wid * _JOBS + job
            batch = t // 144
            rem = t - batch * 144
            htile = rem // 3
            wtile = rem - htile * 3
            row0 = batch * _C + q * _CQ
            h0 = htile * 8
            w0 = wtile * 128
            src_o = o_hbm.at[pl.ds(row0, _CQ), pl.ds(h0, 8), pl.ds(w0, 128)]
            src_l = l_hbm.at[pl.ds(row0, _CQ), pl.ds(h0, 8), pl.ds(w0, 128)]
            return (pltpu.make_async_copy(src_o, obuf.at[slot], isem),
                    pltpu.make_async_copy(src_l, lbuf.at[slot], isem))

        def out_copies(job):
            t = wid * _JOBS + job
            return tuple(
                pltpu.make_async_copy(sbuf.at[r], dst.at[pl.ds(t * 8, 8)],
                                      ssem)
                for r, dst in enumerate(outs))

        def compute(q, slot):
            def jbody(j, _):
                jh = j // 8
                base = (j - jh * 8) * 16

                def ld(buf, c):
                    return buf[slot, c, jh, pl.ds(base, 16)]

                z = jnp.zeros((16,), _F32)
                n = z
                oo = z
                llv = z
                ls = z
                for c in range(_CQ):
                    ov = ld(obuf, c)
                    lv = ld(lbuf, c)
                    n = n + ov * lv
                    oo = oo + ov * ov
                    llv = llv + lv * lv
                    ls = ls + lv
                if q > 0:
                    n = n + sbuf[0, jh, pl.ds(base, 16)]
                    oo = oo + sbuf[1, jh, pl.ds(base, 16)]
                    llv = llv + sbuf[2, jh, pl.ds(base, 16)]
                    ls = ls + sbuf[3, jh, pl.ds(base, 16)]
                sbuf[0, jh, pl.ds(base, 16)] = n
                sbuf[1, jh, pl.ds(base, 16)] = oo
                sbuf[2, jh, pl.ds(base, 16)] = llv
                sbuf[3, jh, pl.ds(base, 16)] = ls
                return 0

            lax.fori_loop(0, 64, jbody, 0)

        def do_job(job, start_next_job):
            for q in range(4):
                slot = q & 1
                if q < 3:
                    for cp in in_copies(job, q + 1, 1 - slot):
                        cp.start()
                elif start_next_job:
                    for cp in in_copies(job + 1, 0, 1 - slot):
                        cp.start()
                for cp in in_copies(job, q, slot):
                    cp.wait()
                if q == 0:
                    for cp in out_copies(job):
                        cp.wait()
                compute(q, slot)
                if q == 3:
                    for cp in out_copies(job):
                        cp.start()

        for cp in in_copies(0, 0, 0):
            cp.start()
        for cp in out_copies(0):
            cp.start()

        def loop_body(job, _):
            do_job(job, True)
            return 0

        lax.fori_loop(0, _JOBS - 1, loop_body, 0)
        do_job(_JOBS - 1, False)
        for cp in out_copies(_JOBS - 1):
            cp.wait()

    return body(o3, l3)


def _acos(x):
    ax = jnp.minimum(jnp.abs(x), 1.0)
    p = jnp.float32(-0.0012624911)
    for c in (0.0066700901, -0.0170881256, 0.0308918810, -0.0501743046,
              0.0889789874, -0.2145988016, 1.5707963050):
        p = p * ax + jnp.float32(c)
    r = jnp.sqrt(1.0 - ax) * p
    return jnp.where(x < 0, jnp.float32(_PI) - r, r)


def _finish_body(n_ref, oo_ref, ll_ref, ls_ref, out_ref, acc_ref):
    i = pl.program_id(0)
    num = n_ref[...]
    oo = oo_ref[...]
    ll = ll_ref[...]
    ls = ls_ref[...]
    mask = ls != 0.0
    den = jnp.sqrt(oo) * jnp.sqrt(ll)
    ratio = jnp.clip(num / jnp.where(mask, den, 1.0), -1.0, 1.0)
    ang = jnp.where(mask, _acos(ratio), 0.0)
    psum = jnp.sum(ang)
    pcnt = jnp.sum(mask.astype(_F32))

    @pl.when(i == 0)
    def _init():
        acc_ref[0] = 0.0
        acc_ref[1] = 0.0

    acc_ref[0] += psum
    acc_ref[1] += pcnt

    @pl.when(i == pl.num_programs(0) - 1)
    def _fin():
        out_ref[0, 0] = acc_ref[0] / acc_ref[1]


def kernel(outputs, labels):
    b, c, h, w = outputs.shape
    o3 = outputs.reshape(b * c, h, w)
    l3 = labels.reshape(b * c, h, w)
    stats = _sc_stats(o3, l3)
    rows = _TILES * 8
    grid = 4
    spec = pl.BlockSpec((rows // grid, 128), lambda i: (i, 0))
    out = pl.pallas_call(
        _finish_body,
        grid=(grid,),
        in_specs=[spec] * 4,
        out_specs=pl.BlockSpec(memory_space=pltpu.SMEM),
        out_shape=jax.ShapeDtypeStruct((1, 1), _F32),
        scratch_shapes=[pltpu.SMEM((2,), _F32)],
    )(*stats)
    return out[0, 0]


# speedup vs baseline: 3.1103x; 1.3997x over previous
import functools

import jax
import jax.numpy as jnp
from jax import lax
from jax.experimental import pallas as pl
from jax.experimental.pallas import tpu as pltpu
from jax.experimental.pallas import tpu_sc as plsc

_F32 = jnp.float32
_PI = 3.141592653589793

_C = 96
_H = 384
_W = 384
_NW = 32
_CQ = _C // 4

_HT2 = 40
_T = 3 * _HT2
_HBLK = 16


def _sc_stats(o3, l3):
    mesh = plsc.VectorSubcoreMesh(core_axis_name="c", subcore_axis_name="s")
    out_t = tuple(
        jax.ShapeDtypeStruct((_T * 8, 128), _F32) for _ in range(4))
    jbase = _T // _NW
    jrem = _T % _NW

    @functools.partial(
        pl.kernel,
        out_type=out_t,
        mesh=mesh,
        compiler_params=pltpu.CompilerParams(use_tc_tiling_on_sc=True),
        scratch_types=[
            pltpu.VMEM((2, _CQ, 8, 128), _F32),
            pltpu.VMEM((2, _CQ, 8, 128), _F32),
            pltpu.VMEM((4, 8, 128), _F32),
            pltpu.SemaphoreType.DMA,
            pltpu.SemaphoreType.DMA,
        ],
    )
    def body(o_hbm, l_hbm, num_o, oo_o, ll_o, ls_o, obuf, lbuf, sbuf,
             isem, ssem):
        wid = lax.axis_index("s") * 2 + lax.axis_index("c")
        outs = (num_o, oo_o, ll_o, ls_o)
        tile0 = wid * jbase + jnp.minimum(wid, jrem)
        njobs = jbase + (wid < jrem).astype(jnp.int32)

        def in_copies(t, q, slot):
            batch = t // 144
            rem = t - batch * 144
            htile = rem // 3
            wtile = rem - htile * 3
            row0 = batch * _C + q * _CQ
            h0 = htile * 8
            w0 = wtile * 128
            src_o = o_hbm.at[pl.ds(row0, _CQ), pl.ds(h0, 8), pl.ds(w0, 128)]
            src_l = l_hbm.at[pl.ds(row0, _CQ), pl.ds(h0, 8), pl.ds(w0, 128)]
            return (pltpu.make_async_copy(src_o, obuf.at[slot], isem),
                    pltpu.make_async_copy(src_l, lbuf.at[slot], isem))

        def out_copies(t):
            return tuple(
                pltpu.make_async_copy(sbuf.at[r], dst.at[pl.ds(t * 8, 8)],
                                      ssem)
                for r, dst in enumerate(outs))

        def compute(q, slot):
            def jbody(j, _):
                jh = j // 8
                base = (j - jh * 8) * 16

                z = jnp.zeros((16,), _F32)
                n = z
                oo = z
                llv = z
                ls = z
                for c in range(_CQ):
                    ov = obuf[slot, c, jh, pl.ds(base, 16)]
                    lv = lbuf[slot, c, jh, pl.ds(base, 16)]
                    n = n + ov * lv
                    oo = oo + ov * ov
                    llv = llv + lv * lv
                    ls = ls + lv
                if q > 0:
                    n = n + sbuf[0, jh, pl.ds(base, 16)]
                    oo = oo + sbuf[1, jh, pl.ds(base, 16)]
                    llv = llv + sbuf[2, jh, pl.ds(base, 16)]
                    ls = ls + sbuf[3, jh, pl.ds(base, 16)]
                sbuf[0, jh, pl.ds(base, 16)] = n
                sbuf[1, jh, pl.ds(base, 16)] = oo
                sbuf[2, jh, pl.ds(base, 16)] = llv
                sbuf[3, jh, pl.ds(base, 16)] = ls
                return 0

            lax.fori_loop(0, 64, jbody, 0)

        def do_job(t, start_next_job):
            for q in range(4):
                slot = q & 1
                if q < 3:
                    for cp in in_copies(t, q + 1, 1 - slot):
                        cp.start()
                elif start_next_job:
                    for cp in in_copies(t + 1, 0, 1 - slot):
                        cp.start()
                for cp in in_copies(t, q, slot):
                    cp.wait()
                if q == 0:
                    for cp in out_copies(t):
                        cp.wait()
                compute(q, slot)
                if q == 3:
                    for cp in out_copies(t):
                        cp.start()

        for cp in in_copies(tile0, 0, 0):
            cp.start()
        for cp in out_copies(tile0):
            cp.start()

        def loop_body(job, _):
            do_job(tile0 + job, True)
            return 0

        lax.fori_loop(0, njobs - 1, loop_body, 0)
        do_job(tile0 + njobs - 1, False)
        for cp in out_copies(tile0 + njobs - 1):
            cp.wait()

    return body(o3, l3)


def _acos(x):
    ax = jnp.minimum(jnp.abs(x), 1.0)
    p = jnp.float32(-0.0012624911)
    for c in (0.0066700901, -0.0170881256, 0.0308918810, -0.0501743046,
              0.0889789874, -0.2145988016, 1.5707963050):
        p = p * ax + jnp.float32(c)
    r = jnp.sqrt(1.0 - ax) * p
    return jnp.where(x < 0, jnp.float32(_PI) - r, r)


def _angles_partial(num, oo, ll, ls):
    mask = ls != 0.0
    den = jnp.sqrt(oo) * jnp.sqrt(ll)
    ratio = jnp.clip(num / jnp.where(mask, den, 1.0), -1.0, 1.0)
    ang = jnp.where(mask, _acos(ratio), 0.0)
    return jnp.sum(ang), jnp.sum(mask.astype(_F32))


def _tc_body(o_ref, l_ref, out_ref):
    i = pl.program_id(0)
    o = o_ref[0]
    l = l_ref[0]
    num = jnp.sum(o * l, axis=0)
    oo = jnp.sum(o * o, axis=0)
    ll = jnp.sum(l * l, axis=0)
    ls = jnp.sum(l, axis=0)
    psum, pcnt = _angles_partial(num, oo, ll, ls)

    @pl.when(i == 0)
    def _init():
        out_ref[0] = 0.0
        out_ref[1] = 0.0

    out_ref[0] += psum
    out_ref[1] += pcnt


def _finish_body(n_ref, oo_ref, ll_ref, ls_ref, tc_ref, out_ref, acc_ref):
    i = pl.program_id(0)
    psum, pcnt = _angles_partial(n_ref[...], oo_ref[...], ll_ref[...],
                                 ls_ref[...])

    @pl.when(i == 0)
    def _init():
        acc_ref[0] = 0.0
        acc_ref[1] = 0.0

    acc_ref[0] += psum
    acc_ref[1] += pcnt

    @pl.when(i == pl.num_programs(0) - 1)
    def _fin():
        out_ref[0, 0] = (acc_ref[0] + tc_ref[0]) / (acc_ref[1] + tc_ref[1])


def kernel(outputs, labels):
    b, c, h, w = outputs.shape
    o3 = outputs.reshape(b * c, h, w)
    l3 = labels.reshape(b * c, h, w)
    stats = _sc_stats(o3, l3)

    nh0 = (_H - 8 * _HT2) // _HBLK
    nh1 = _H // _HBLK
    h0_off = (8 * _HT2) // _HBLK

    def imap(i):
        in_b1 = i >= nh0
        return (in_b1.astype(jnp.int32), 0,
                jnp.where(in_b1, i - nh0, i + h0_off), 0)

    spec = pl.BlockSpec((1, c, _HBLK, w), imap)
    tc_part = pl.pallas_call(
        _tc_body,
        grid=(nh0 + nh1,),
        in_specs=[spec, spec],
        out_specs=pl.BlockSpec(memory_space=pltpu.SMEM),
        out_shape=jax.ShapeDtypeStruct((2,), _F32),
    )(outputs, labels)

    rows = _T * 8
    grid = 4
    sspec = pl.BlockSpec((rows // grid, 128), lambda i: (i, 0))
    out = pl.pallas_call(
        _finish_body,
        grid=(grid,),
        in_specs=[sspec] * 4 + [pl.BlockSpec(memory_space=pltpu.SMEM)],
        out_specs=pl.BlockSpec(memory_space=pltpu.SMEM),
        out_shape=jax.ShapeDtypeStruct((1, 1), _F32),
        scratch_shapes=[pltpu.SMEM((2,), _F32)],
    )(*stats, tc_part)
    return out[0, 0]
